# baseline (speedup 1.0000x reference)
import functools

import jax
import jax.numpy as jnp
from jax import lax
from jax.experimental import pallas as pl
from jax.experimental.pallas import tpu as pltpu
from jax.experimental.pallas import tpu_sc as plsc

NUM_BINS = 1000000
SALTS = (1, 2)
L = 16
CHUNK = 128
W = 128


def _hash_mod_bins(h):
    hi = plsc.bitcast(h, jnp.int32)
    hf = hi.astype(jnp.float32)
    hf = jnp.where(hi < 0, hf + jnp.float32(4294967296.0), hf)
    q = (hf * jnp.float32(1.0 / NUM_BINS)).astype(jnp.int32)
    r = hi - q * jnp.int32(NUM_BINS)
    r = jnp.where(r < 0, r + jnp.int32(NUM_BINS), r)
    r = jnp.where(r >= jnp.int32(NUM_BINS), r - jnp.int32(NUM_BINS), r)
    return r


def _hash_ids(ids, salt_const, offset):
    h = plsc.bitcast(ids, jnp.uint32)
    h = h * jnp.uint32(2654435761)
    h = h ^ jnp.uint32(salt_const)
    h = h ^ (h >> 16)
    h = h * jnp.uint32(0x85EBCA6B)
    h = h ^ (h >> 13)
    h = h * jnp.uint32(0xC2B2AE35)
    h = h ^ (h >> 16)
    return _hash_mod_bins(h) + jnp.int32(offset)


@functools.partial(jax.jit, static_argnames=("n", "d"))
def _sc_lookup(ids_flat, table_w, n, d):
    qd = W // d
    qs = qd.bit_length() - 1
    ds_ = d.bit_length() - 1
    opr = CHUNK // qd
    info = plsc.get_sparse_core_info()
    nc, ns = info.num_cores, info.num_subcores
    nw = nc * ns
    per_w = n // nw
    n_chunks = per_w // CHUNK
    n_half = n_chunks // 2
    mesh = plsc.VectorSubcoreMesh(core_axis_name="c", subcore_axis_name="s")
    salt_consts = [(s * 0x9E3779B9) & 0xFFFFFFFF for s in SALTS]

    def buf_set():
        return [
            pltpu.VMEM((CHUNK,), jnp.int32),
            pltpu.VMEM((CHUNK,), jnp.int32),
            pltpu.VMEM((CHUNK,), jnp.int32),
            pltpu.VMEM((CHUNK,), jnp.int32),
            pltpu.VMEM((CHUNK,), jnp.int32),
            pltpu.VMEM((CHUNK, W), jnp.float32),
            pltpu.VMEM((CHUNK, W), jnp.float32),
            pltpu.VMEM((opr, W), jnp.float32),
        ]

    @functools.partial(
        pl.kernel,
        mesh=mesh,
        compiler_params=pltpu.CompilerParams(needs_layout_passes=False),
        out_type=jax.ShapeDtypeStruct((n // qd, W), jnp.float32),
        scratch_types=buf_set() + buf_set() + [pltpu.SemaphoreType.DMA] * 6,
    )
    def k(ids_hbm, table_hbm, out_hbm,
          ids_a, g0_a, g1_a, m0_a, m1_a, f0_a, f1_a, ob_a,
          ids_b, g0_b, g1_b, m0_b, m1_b, f0_b, f1_b, ob_b,
          si0, si1, sg0, sg1, so0, so1):
        wid = lax.axis_index("s") * nc + lax.axis_index("c")
        base = wid * per_w
        obase = base >> qs
        ids_v = (ids_a, ids_b)
        g0_v = (g0_a, g0_b)
        g1_v = (g1_a, g1_b)
        m0_v = (m0_a, m0_b)
        m1_v = (m1_a, m1_b)
        f0_v = (f0_a, f0_b)
        f1_v = (f1_a, f1_b)
        ob_v = (ob_a, ob_b)
        sem_i = (si0, si1)
        sem_g = (sg0, sg1)
        sem_o = (so0, so1)

        def hash_chunk(b):
            for j in range(CHUNK // L):
                sl = pl.ds(j * L, L)
                ids = ids_v[b][sl]
                r0 = _hash_ids(ids, salt_consts[0], 0)
                r1 = _hash_ids(ids, salt_consts[1], NUM_BINS)
                g0_v[b][sl] = r0 >> qs
                m0_v[b][sl] = (r0 & (qd - 1)) << ds_
                g1_v[b][sl] = r1 >> qs
                m1_v[b][sl] = (r1 & (qd - 1)) << ds_

        def gather_cps(b):
            return [
                pltpu.make_async_copy(table_hbm.at[g0_v[b]], f0_v[b],
                                      sem_g[b]),
                pltpu.make_async_copy(table_hbm.at[g1_v[b]], f1_v[b],
                                      sem_g[b]),
            ]

        def out_cp(b, ci):
            start = pl.multiple_of(obase + ci * opr, opr)
            return pltpu.make_async_copy(
                ob_v[b], out_hbm.at[pl.ds(start, opr)], sem_o[b])

        def extract_sum(b):
            lane = lax.iota(jnp.int32, L)
            orow0 = lane >> qs
            ocol0 = (lane & (qd - 1)) << ds_

            def rb_body(rb, carry):
                rsl = pl.ds(rb * L, L)
                rows = lane + rb * L
                c0 = m0_v[b][rsl]
                c1 = m1_v[b][rsl]
                orow = orow0 + rb * (L >> qs)
                for c in range(d):
                    v0 = plsc.load_gather(f0_v[b], [rows, c0 + c])
                    v1 = plsc.load_gather(f1_v[b], [rows, c1 + c])
                    plsc.store_scatter(ob_v[b], [orow, ocol0 + c], v0 + v1)
                return carry

            lax.fori_loop(0, CHUNK // L, rb_body, 0)

        pltpu.sync_copy(ids_hbm.at[pl.ds(base, CHUNK)], ids_v[0])
        pltpu.async_copy(ids_hbm.at[pl.ds(base + CHUNK, CHUNK)], ids_v[1],
                         sem_i[1])
        hash_chunk(0)
        for cp in gather_cps(0):
            cp.start()

        def body(i, carry):
            for b in (0, 1):
                ci = 2 * i + b
                other = 1 - b
                for cp in gather_cps(b):
                    cp.wait()

                @pl.when(ci + 1 < n_chunks)
                def _stage_next():
                    pltpu.make_async_copy(
                        ids_hbm.at[pl.ds(base + (ci + 1) * CHUNK, CHUNK)],
                        ids_v[other], sem_i[other]).wait()
                    hash_chunk(other)

                    @pl.when(ci >= 1)
                    def _drain_prev_store():
                        out_cp(other, ci).wait()

                    for cp in gather_cps(other):
                        cp.start()

                    @pl.when(ci + 2 < n_chunks)
                    def _prefetch_ids():
                        pltpu.async_copy(
                            ids_hbm.at[pl.ds(base + (ci + 2) * CHUNK, CHUNK)],
                            ids_v[b], sem_i[b])

                extract_sum(b)
                out_cp(b, ci).start()
            return carry

        lax.fori_loop(0, n_half, body, 0)
        out_cp(0, 0).wait()
        out_cp(1, 0).wait()

    return k(ids_flat, table_w)


def kernel(inputs, table):
    b, f = inputs.shape
    d = table.shape[1]
    n = b * f
    ids_flat = inputs.reshape(n)
    table_w = table.reshape(table.shape[0] * d // W, W)
    info = plsc.get_sparse_core_info()
    grain = info.num_cores * info.num_subcores * CHUNK * 2
    n_pad = ((n + grain - 1) // grain) * grain
    if n_pad != n:
        ids_flat = jnp.pad(ids_flat, (0, n_pad - n))
    out = _sc_lookup(ids_flat, table_w, n_pad, d)
    return out.reshape(n_pad, d)[:n].reshape(b, f, d)
